# baseline (speedup 1.0000x reference)
import functools

import jax
import jax.numpy as jnp
from jax import lax
from jax.experimental import pallas as pl
from jax.experimental.pallas import tpu as pltpu
from jax.experimental.pallas import tpu_sc as plsc

GRP = 8
C2 = 2048
NG = C2 // GRP
NBSZ = 256
UPD = 512
SC_G = 125
SC_NI = 8
SP_ROUND = 256000
XW = 16
H = 64


def _cdiv(a, b):
    return (a + b - 1) // b


def _sc_gather(table, idx):
    ep = idx.shape[0]
    w, dt = table.shape[1], table.dtype
    info = plsc.get_sparse_core_info()
    nc, ns = info.num_cores, info.num_subcores
    nw = nc * ns
    rows_pw = ep // (nw * SC_G)
    n_outer = rows_pw // SC_NI
    idx2d = idx.reshape(ep // SC_G, SC_G)
    mesh = plsc.VectorSubcoreMesh(core_axis_name="c", subcore_axis_name="s")

    @functools.partial(
        pl.kernel,
        mesh=mesh,
        out_type=jax.ShapeDtypeStruct((ep // SC_G, SC_G, w), dt),
        scratch_types=[
            pltpu.VMEM((SC_NI, SC_G), jnp.int32),
            pltpu.VMEM((SC_NI, SC_G, w), dt),
            pltpu.SemaphoreType.DMA,
        ],
        compiler_params=pltpu.CompilerParams(use_tc_tiling_on_sc=False),
    )
    def gather_kernel(table_hbm, idx_hbm, out_hbm, idx_v, rows_v, sem):
        wid = lax.axis_index("s") * nc + lax.axis_index("c")
        base = wid * rows_pw

        def outer(i, carry):
            row0 = base + i * SC_NI
            pltpu.sync_copy(idx_hbm.at[pl.ds(row0, SC_NI)], idx_v)
            descs = []
            for j in range(SC_NI):
                descs.append(pltpu.async_copy(
                    table_hbm.at[idx_v.at[j]], rows_v.at[j], sem))
            for d in descs:
                d.wait()
            pltpu.sync_copy(rows_v, out_hbm.at[pl.ds(row0, SC_NI)])
            return carry

        lax.fori_loop(0, n_outer, outer, 0)

    return gather_kernel(table, idx2d).reshape(ep, w)


def _agg_body(cnb_r, cg0_r, cf_r, wf_r, xg_r, ea_r, val_r, gn_r, g0_r, g1_r,
              wxbd_r, webd_r, b1t_r, w1bd_r, b2t_r, vex_r, agg_r,
              *, ng, grp, nbsz):
    c = pl.program_id(0)

    @pl.when(wf_r[c] == 1)
    def _work():
        gc0 = cg0_r[c] * ng
        bf = jnp.bfloat16
        h = jnp.dot(xg_r[...].astype(bf), wxbd_r[...],
                    preferred_element_type=jnp.float32)
        h = h + jnp.dot(ea_r[...].astype(bf), webd_r[...],
                        preferred_element_type=jnp.float32)
        h = jnp.maximum(h + b1t_r[...], 0.0)
        h = jnp.dot(h.astype(bf), w1bd_r[...],
                    preferred_element_type=jnp.float32)
        h = jnp.maximum(h + b2t_r[...], 0.0)
        vl = jnp.dot(val_r[...], vex_r[...],
                     preferred_element_type=jnp.float32)
        h = h * vl
        m = h[:, 0:H]
        for j in range(1, grp):
            m = jnp.maximum(m, h[:, j * H:(j + 1) * H])
        gn = gn_r[...]
        k = 1
        while k < ng:
            ms = jnp.concatenate([jnp.zeros((k, H), jnp.float32), m[:-k]], axis=0)
            gs = jnp.concatenate([jnp.full((k, 1), -1, jnp.int32), gn[:-k]], axis=0)
            m = jnp.maximum(m, jnp.where(gs == gn, ms, 0.0))
            k *= 2
        g0 = g0_r[...]
        g1 = g1_r[...]
        pos = jnp.clip(g1 - 1, gc0, gc0 + ng - 1) - gc0
        has = (g1 > gc0) & (g0 < gc0 + ng) & (g1 > g0)
        lanes = lax.broadcasted_iota(jnp.int32, (nbsz, ng), 1)
        sel = jnp.where((lanes == pos) & has, 1.0, 0.0)
        contrib = jnp.dot(sel, m, preferred_element_type=jnp.float32)

        @pl.when(cf_r[c] == 1)
        def _():
            agg_r[...] = contrib

        @pl.when(cf_r[c] == 0)
        def _():
            agg_r[...] = jnp.maximum(agg_r[...], contrib)


def _edge_agg(xg, ea, val, gn2d, g0, g1, wxbd, webd, b1t, w1bd, b2t, vex,
              cnb, cg0, cf, wf, n_pad, interpret=False):
    maxc = cnb.shape[0]
    grp, nbsz = GRP, NBSZ
    ng = C2 // grp
    grid_spec = pltpu.PrefetchScalarGridSpec(
        num_scalar_prefetch=4,
        grid=(maxc,),
        in_specs=[
            pl.BlockSpec((ng, grp * XW), lambda c, cnb, cg0, cf, wf: (cg0[c], 0)),
            pl.BlockSpec((ng, grp * 4), lambda c, cnb, cg0, cf, wf: (cg0[c], 0)),
            pl.BlockSpec((ng, grp), lambda c, cnb, cg0, cf, wf: (cg0[c], 0)),
            pl.BlockSpec((ng, 1), lambda c, cnb, cg0, cf, wf: (cg0[c], 0)),
            pl.BlockSpec((nbsz, 1), lambda c, cnb, cg0, cf, wf: (cnb[c], 0)),
            pl.BlockSpec((nbsz, 1), lambda c, cnb, cg0, cf, wf: (cnb[c], 0)),
            pl.BlockSpec((grp * XW, grp * H), lambda c, cnb, cg0, cf, wf: (0, 0)),
            pl.BlockSpec((grp * 4, grp * H), lambda c, cnb, cg0, cf, wf: (0, 0)),
            pl.BlockSpec((1, grp * H), lambda c, cnb, cg0, cf, wf: (0, 0)),
            pl.BlockSpec((grp * H, grp * H), lambda c, cnb, cg0, cf, wf: (0, 0)),
            pl.BlockSpec((1, grp * H), lambda c, cnb, cg0, cf, wf: (0, 0)),
            pl.BlockSpec((grp, grp * H), lambda c, cnb, cg0, cf, wf: (0, 0)),
        ],
        out_specs=pl.BlockSpec((nbsz, H), lambda c, cnb, cg0, cf, wf: (cnb[c], 0)),
    )
    return pl.pallas_call(
        functools.partial(_agg_body, ng=ng, grp=grp, nbsz=nbsz),
        grid_spec=grid_spec,
        out_shape=jax.ShapeDtypeStruct((n_pad, H), jnp.float32),
        compiler_params=pltpu.CompilerParams(
            dimension_semantics=("arbitrary",)),
        interpret=interpret,
    )(cnb, cg0, cf, wf, xg, ea, val, gn2d, g0, g1,
      wxbd, webd, b1t, w1bd, b2t, vex)


def _upd_body(x_r, agg_r, w2ax_r, w2aa_r, b2a_r, w2b_r, b2b_r, out_r):
    x = x_r[...]
    t = jnp.dot(x, w2ax_r[...], preferred_element_type=jnp.float32)
    t = t + jnp.dot(agg_r[...], w2aa_r[...], preferred_element_type=jnp.float32)
    t = jnp.maximum(t + b2a_r[...], 0.0)
    comb = jnp.dot(t, w2b_r[...], preferred_element_type=jnp.float32) + b2b_r[...]
    nor = jnp.sum(comb * comb, axis=1, keepdims=True)
    comb = comb * (1.0 / jnp.maximum(1.0, jnp.sqrt(nor)))
    z = jnp.zeros((x.shape[0], XW - 8), jnp.float32)
    out_r[...] = jnp.concatenate([comb[:, :4], x[:, :4], z], axis=1)


def _node_update(x_pad, agg, w2ax, w2aa, b2a2, w2b8, b2b8, interpret=False):
    n_pad = x_pad.shape[0]
    upd = UPD
    grid = (n_pad // upd,)
    return pl.pallas_call(
        _upd_body,
        grid=grid,
        in_specs=[
            pl.BlockSpec((upd, XW), lambda u: (u, 0)),
            pl.BlockSpec((upd, H), lambda u: (u, 0)),
            pl.BlockSpec((XW, 32), lambda u: (0, 0)),
            pl.BlockSpec((H, 32), lambda u: (0, 0)),
            pl.BlockSpec((1, 32), lambda u: (0, 0)),
            pl.BlockSpec((32, 8), lambda u: (0, 0)),
            pl.BlockSpec((1, 8), lambda u: (0, 0)),
        ],
        out_specs=pl.BlockSpec((upd, XW), lambda u: (u, 0)),
        out_shape=jax.ShapeDtypeStruct((n_pad, XW), jnp.float32),
        interpret=interpret,
    )(x_pad, agg, w2ax, w2aa, b2a2, w2b8, b2b8)


def kernel(x, edge_index, edge_attr, W1a, b1a, W1b, b1b, W2a, b2a, W2b, b2b):
    n, f = x.shape
    e = edge_attr.shape[0]
    i32, f32 = jnp.int32, jnp.float32
    bf = jnp.bfloat16
    src = edge_index[0].astype(i32)
    dst = edge_index[1].astype(i32)

    nblk = _cdiv(n, NBSZ)
    nblk = _cdiv(nblk, 4) * 4
    n_pad = nblk * NBSZ
    sp = _cdiv(e + (GRP - 1) * min(n, e), SP_ROUND) * SP_ROUND
    gmax = sp // GRP
    maxc = sp // C2 + nblk

    qr = jnp.arange(n_pad + 1, dtype=i32)
    keys = jnp.concatenate([dst * 2 + 1, qr * 2])
    inv2 = jnp.argsort(jnp.argsort(keys)).astype(i32)
    rowptr = inv2[e:] - qr
    erank = inv2[:e] - (dst + 1)
    perm = jnp.argsort(erank).astype(i32)
    deg = rowptr[1:] - rowptr[:n_pad]
    ngrp = (deg + GRP - 1) // GRP
    gstart = jnp.concatenate([jnp.zeros((1,), i32),
                              jnp.cumsum(ngrp).astype(i32)])
    qg = jnp.arange(gmax, dtype=i32)
    gk = jnp.concatenate([gstart[1:] * 2, qg * 2 + 1])
    inv3 = jnp.argsort(jnp.argsort(gk)).astype(i32)
    gnode = inv3[n_pad:] - qg
    base = (rowptr[gnode] - GRP * gstart[gnode]
            + GRP * jnp.arange(gmax, dtype=i32))
    e2d = base[:, None] + jnp.arange(GRP, dtype=i32)[None, :]
    lim = rowptr[jnp.minimum(gnode + 1, n_pad)]
    validm = e2d < lim[:, None]
    s2e = jnp.where(validm, e2d, e).reshape(sp)
    perm_pad = jnp.concatenate([perm, jnp.full((1,), e, i32)])
    src_pad = jnp.concatenate([src, jnp.full((1,), n, i32)])
    ea_pad = jnp.concatenate([edge_attr.astype(f32), jnp.zeros((1, 4), f32)])
    perm_t = jnp.zeros((e + 1, XW), f32).at[:, 0].set(
        lax.bitcast_convert_type(perm_pad, f32))
    pe = lax.bitcast_convert_type(_sc_gather(perm_t, s2e)[:, 0], i32)
    edge_t = jnp.zeros((e + 1, XW), f32).at[:, :4].set(ea_pad).at[:, 4].set(
        lax.bitcast_convert_type(src_pad, f32))
    rows = _sc_gather(edge_t, pe)
    slot_src = lax.bitcast_convert_type(rows[:, 4], i32)
    ea_g = rows[:, :4].reshape(gmax, GRP * 4)
    val_g = validm.astype(f32)
    gn2d = gnode.reshape(gmax, 1)
    g0 = gstart[:n_pad].reshape(n_pad, 1)
    g1 = gstart[1:].reshape(n_pad, 1)

    gb0 = gstart[0:n_pad:NBSZ]
    gb1 = gstart[NBSZ:n_pad + 1:NBSZ]
    ngc = C2 // GRP
    cs = gb0 // ngc
    ce = (gb1 + ngc - 1) // ngc
    nch = jnp.maximum(ce - cs, 1)
    off = jnp.cumsum(nch) - nch
    total_chunks = off[-1] + nch[-1]
    cid = jnp.arange(maxc, dtype=i32)
    ck = jnp.concatenate([off.astype(i32) * 2, cid * 2 + 1])
    inv4 = jnp.argsort(jnp.argsort(ck)).astype(i32)
    cnb = inv4[nblk:] - cid - 1
    cnb = jnp.clip(cnb, 0, nblk - 1)
    within = cid - off[cnb]
    cg0 = jnp.clip(cs[cnb] + within, 0, gmax // ngc - 1).astype(i32)
    cf = (within == 0).astype(i32)
    wf = (cid < total_chunks).astype(i32)

    wx = jnp.zeros((XW, H), f32).at[:f].set(W1a[:f])
    we = W1a[f:]
    eye = jnp.eye(GRP, dtype=f32)
    wxbd = jnp.kron(eye, wx).astype(bf)
    webd = jnp.kron(eye, we).astype(bf)
    w1bd = jnp.kron(eye, W1b).astype(bf)
    b1t = jnp.tile(b1a.reshape(1, H), (1, GRP))
    b2t = jnp.tile(b1b.reshape(1, H), (1, GRP))
    vex = jnp.kron(eye, jnp.ones((1, H), f32))
    w2ax = jnp.zeros((XW, 32), f32).at[:f].set(W2a[:f])
    w2aa = W2a[f:]
    b2a2 = b2a.reshape(1, 32)
    w2b8 = jnp.zeros((32, 8), f32).at[:, :4].set(W2b)
    b2b8 = jnp.zeros((1, 8), f32).at[0, :4].set(b2b)

    x_pad = jnp.zeros((n_pad, XW), f32).at[:n, :f].set(x)
    for _ in range(3):
        xj = _sc_gather(x_pad, slot_src)
        xg = xj.reshape(gmax, GRP * XW)
        agg = _edge_agg(xg, ea_g, val_g, gn2d, g0, g1, wxbd, webd, b1t,
                        w1bd, b2t, vex, cnb, cg0, cf, wf, n_pad)
        x_pad = _node_update(x_pad, agg, w2ax, w2aa, b2a2, w2b8, b2b8)
    return x_pad[:n, :f]


# speedup vs baseline: 1.3804x; 1.3804x over previous
import functools

import jax
import jax.numpy as jnp
from jax import lax
from jax.experimental import pallas as pl
from jax.experimental.pallas import tpu as pltpu
from jax.experimental.pallas import tpu_sc as plsc

GRP = 8
C2 = 2048
NG = C2 // GRP
NBSZ = 256
UPD = 512
SC_G = 125
SC_NI = 8
SP_ROUND = 256000
XW = 16
H = 64


def _cdiv(a, b):
    return (a + b - 1) // b


def _sc_gather(table, idx):
    ep = idx.shape[0]
    w, dt = table.shape[1], table.dtype
    info = plsc.get_sparse_core_info()
    nc, ns = info.num_cores, info.num_subcores
    nw = nc * ns
    rows_pw = ep // (nw * SC_G)
    n_outer = rows_pw // SC_NI
    idx2d = idx.reshape(ep // SC_G, SC_G)
    mesh = plsc.VectorSubcoreMesh(core_axis_name="c", subcore_axis_name="s")

    @functools.partial(
        pl.kernel,
        mesh=mesh,
        out_type=jax.ShapeDtypeStruct((ep // SC_G, SC_G, w), dt),
        scratch_types=[
            pltpu.VMEM((SC_NI, SC_G), jnp.int32),
            pltpu.VMEM((SC_NI, SC_G, w), dt),
            pltpu.SemaphoreType.DMA,
        ],
        compiler_params=pltpu.CompilerParams(use_tc_tiling_on_sc=False),
    )
    def gather_kernel(table_hbm, idx_hbm, out_hbm, idx_v, rows_v, sem):
        wid = lax.axis_index("s") * nc + lax.axis_index("c")
        base = wid * rows_pw

        def outer(i, carry):
            row0 = base + i * SC_NI
            pltpu.sync_copy(idx_hbm.at[pl.ds(row0, SC_NI)], idx_v)
            descs = []
            for j in range(SC_NI):
                descs.append(pltpu.async_copy(
                    table_hbm.at[idx_v.at[j]], rows_v.at[j], sem))
            for d in descs:
                d.wait()
            pltpu.sync_copy(rows_v, out_hbm.at[pl.ds(row0, SC_NI)])
            return carry

        lax.fori_loop(0, n_outer, outer, 0)

    return gather_kernel(table, idx2d).reshape(ep, w)


def _agg_body(cnb_r, cg0_r, cf_r, wf_r, xg_r, ea_r, val_r, gn_r, g0_r, g1_r,
              wxbd_r, webd_r, b1t_r, w1bd_r, b2t_r, vex_r, agg_r,
              *, ng, grp, nbsz):
    c = pl.program_id(0)

    @pl.when(wf_r[c] == 1)
    def _work():
        gc0 = cg0_r[c] * ng
        bf = jnp.bfloat16
        h = jnp.dot(xg_r[...].astype(bf), wxbd_r[...],
                    preferred_element_type=jnp.float32)
        h = h + jnp.dot(ea_r[...].astype(bf), webd_r[...],
                        preferred_element_type=jnp.float32)
        h = jnp.maximum(h + b1t_r[...], 0.0)
        h = jnp.dot(h.astype(bf), w1bd_r[...],
                    preferred_element_type=jnp.float32)
        h = jnp.maximum(h + b2t_r[...], 0.0)
        vl = jnp.dot(val_r[...], vex_r[...],
                     preferred_element_type=jnp.float32)
        h = h * vl
        m = h[:, 0:H]
        for j in range(1, grp):
            m = jnp.maximum(m, h[:, j * H:(j + 1) * H])
        gn = gn_r[...]
        k = 1
        while k < ng:
            ms = jnp.concatenate([jnp.zeros((k, H), jnp.float32), m[:-k]], axis=0)
            gs = jnp.concatenate([jnp.full((k, 1), -1, jnp.int32), gn[:-k]], axis=0)
            m = jnp.maximum(m, jnp.where(gs == gn, ms, 0.0))
            k *= 2
        g0 = g0_r[...]
        g1 = g1_r[...]
        pos = jnp.clip(g1 - 1, gc0, gc0 + ng - 1) - gc0
        has = (g1 > gc0) & (g0 < gc0 + ng) & (g1 > g0)
        lanes = lax.broadcasted_iota(jnp.int32, (nbsz, ng), 1)
        sel = jnp.where((lanes == pos) & has, 1.0, 0.0)
        contrib = jnp.dot(sel, m, preferred_element_type=jnp.float32)

        @pl.when(cf_r[c] == 1)
        def _():
            agg_r[...] = contrib

        @pl.when(cf_r[c] == 0)
        def _():
            agg_r[...] = jnp.maximum(agg_r[...], contrib)


def _edge_agg(xg, ea, val, gn2d, g0, g1, wxbd, webd, b1t, w1bd, b2t, vex,
              cnb, cg0, cf, wf, n_pad, interpret=False):
    maxc = cnb.shape[0]
    grp, nbsz = GRP, NBSZ
    ng = C2 // grp
    grid_spec = pltpu.PrefetchScalarGridSpec(
        num_scalar_prefetch=4,
        grid=(maxc,),
        in_specs=[
            pl.BlockSpec((ng, grp * XW), lambda c, cnb, cg0, cf, wf: (cg0[c], 0)),
            pl.BlockSpec((ng, grp * 4), lambda c, cnb, cg0, cf, wf: (cg0[c], 0)),
            pl.BlockSpec((ng, grp), lambda c, cnb, cg0, cf, wf: (cg0[c], 0)),
            pl.BlockSpec((ng, 1), lambda c, cnb, cg0, cf, wf: (cg0[c], 0)),
            pl.BlockSpec((nbsz, 1), lambda c, cnb, cg0, cf, wf: (cnb[c], 0)),
            pl.BlockSpec((nbsz, 1), lambda c, cnb, cg0, cf, wf: (cnb[c], 0)),
            pl.BlockSpec((grp * XW, grp * H), lambda c, cnb, cg0, cf, wf: (0, 0)),
            pl.BlockSpec((grp * 4, grp * H), lambda c, cnb, cg0, cf, wf: (0, 0)),
            pl.BlockSpec((1, grp * H), lambda c, cnb, cg0, cf, wf: (0, 0)),
            pl.BlockSpec((grp * H, grp * H), lambda c, cnb, cg0, cf, wf: (0, 0)),
            pl.BlockSpec((1, grp * H), lambda c, cnb, cg0, cf, wf: (0, 0)),
            pl.BlockSpec((grp, grp * H), lambda c, cnb, cg0, cf, wf: (0, 0)),
        ],
        out_specs=pl.BlockSpec((nbsz, H), lambda c, cnb, cg0, cf, wf: (cnb[c], 0)),
    )
    return pl.pallas_call(
        functools.partial(_agg_body, ng=ng, grp=grp, nbsz=nbsz),
        grid_spec=grid_spec,
        out_shape=jax.ShapeDtypeStruct((n_pad, H), jnp.float32),
        compiler_params=pltpu.CompilerParams(
            dimension_semantics=("arbitrary",)),
        interpret=interpret,
    )(cnb, cg0, cf, wf, xg, ea, val, gn2d, g0, g1,
      wxbd, webd, b1t, w1bd, b2t, vex)


def _upd_body(x_r, agg_r, w2ax_r, w2aa_r, b2a_r, w2b_r, b2b_r, out_r):
    x = x_r[...]
    t = jnp.dot(x, w2ax_r[...], preferred_element_type=jnp.float32)
    t = t + jnp.dot(agg_r[...], w2aa_r[...], preferred_element_type=jnp.float32)
    t = jnp.maximum(t + b2a_r[...], 0.0)
    comb = jnp.dot(t, w2b_r[...], preferred_element_type=jnp.float32) + b2b_r[...]
    nor = jnp.sum(comb * comb, axis=1, keepdims=True)
    comb = comb * (1.0 / jnp.maximum(1.0, jnp.sqrt(nor)))
    z = jnp.zeros((x.shape[0], XW - 8), jnp.float32)
    out_r[...] = jnp.concatenate([comb[:, :4], x[:, :4], z], axis=1)


def _node_update(x_pad, agg, w2ax, w2aa, b2a2, w2b8, b2b8, interpret=False):
    n_pad = x_pad.shape[0]
    upd = UPD
    grid = (n_pad // upd,)
    return pl.pallas_call(
        _upd_body,
        grid=grid,
        in_specs=[
            pl.BlockSpec((upd, XW), lambda u: (u, 0)),
            pl.BlockSpec((upd, H), lambda u: (u, 0)),
            pl.BlockSpec((XW, 32), lambda u: (0, 0)),
            pl.BlockSpec((H, 32), lambda u: (0, 0)),
            pl.BlockSpec((1, 32), lambda u: (0, 0)),
            pl.BlockSpec((32, 8), lambda u: (0, 0)),
            pl.BlockSpec((1, 8), lambda u: (0, 0)),
        ],
        out_specs=pl.BlockSpec((upd, XW), lambda u: (u, 0)),
        out_shape=jax.ShapeDtypeStruct((n_pad, XW), jnp.float32),
        interpret=interpret,
    )(x_pad, agg, w2ax, w2aa, b2a2, w2b8, b2b8)


def kernel(x, edge_index, edge_attr, W1a, b1a, W1b, b1b, W2a, b2a, W2b, b2b):
    n, f = x.shape
    e = edge_attr.shape[0]
    i32, f32 = jnp.int32, jnp.float32
    bf = jnp.bfloat16
    src = edge_index[0].astype(i32)
    dst = edge_index[1].astype(i32)

    nblk = _cdiv(n, NBSZ)
    nblk = _cdiv(nblk, 4) * 4
    n_pad = nblk * NBSZ
    sp = _cdiv(e + (GRP - 1) * min(n, e), SP_ROUND) * SP_ROUND
    gmax = sp // GRP
    maxc = sp // C2 + nblk

    qr = jnp.arange(n_pad + 1, dtype=i32)
    keys = jnp.concatenate([dst * 2 + 1, qr * 2])
    inv2 = jnp.argsort(jnp.argsort(keys)).astype(i32)
    rowptr = inv2[e:] - qr
    erank = inv2[:e] - (dst + 1)
    perm = jnp.argsort(erank).astype(i32)
    deg = rowptr[1:] - rowptr[:n_pad]
    ngrp = (deg + GRP - 1) // GRP
    gstart = jnp.concatenate([jnp.zeros((1,), i32),
                              jnp.cumsum(ngrp).astype(i32)])
    qg = jnp.arange(gmax, dtype=i32)
    gk = jnp.concatenate([gstart[1:] * 2, qg * 2 + 1])
    inv3 = jnp.argsort(jnp.argsort(gk)).astype(i32)
    gnode = inv3[n_pad:] - qg
    base = (rowptr[gnode] - GRP * gstart[gnode]
            + GRP * jnp.arange(gmax, dtype=i32))
    e2d = base[:, None] + jnp.arange(GRP, dtype=i32)[None, :]
    lim = rowptr[jnp.minimum(gnode + 1, n_pad)]
    validm = e2d < lim[:, None]
    epad = 8192
    vflat = validm.reshape(sp)
    spread = jnp.arange(sp, dtype=i32) % epad
    s2e = jnp.where(vflat, e2d.reshape(sp), e + spread)
    perm_t = jnp.zeros((e + epad, XW), f32).at[:e, 0].set(
        lax.bitcast_convert_type(perm, f32))
    pe = lax.bitcast_convert_type(_sc_gather(perm_t, s2e)[:, 0], i32)
    pe = jnp.where(vflat, pe, e + spread)
    edge_t = jnp.zeros((e + epad, XW), f32).at[:e, :4].set(
        edge_attr.astype(f32)).at[:e, 4].set(
        lax.bitcast_convert_type(src, f32))
    rows = _sc_gather(edge_t, pe)
    slot_src = jnp.where(vflat,
                         lax.bitcast_convert_type(rows[:, 4], i32),
                         n_pad + spread)
    ea_g = rows[:, :4].reshape(gmax, GRP * 4)
    val_g = validm.astype(f32)
    gn2d = gnode.reshape(gmax, 1)
    g0 = gstart[:n_pad].reshape(n_pad, 1)
    g1 = gstart[1:].reshape(n_pad, 1)

    gb0 = gstart[0:n_pad:NBSZ]
    gb1 = gstart[NBSZ:n_pad + 1:NBSZ]
    ngc = C2 // GRP
    cs = gb0 // ngc
    ce = (gb1 + ngc - 1) // ngc
    nch = jnp.maximum(ce - cs, 1)
    off = jnp.cumsum(nch) - nch
    total_chunks = off[-1] + nch[-1]
    cid = jnp.arange(maxc, dtype=i32)
    ck = jnp.concatenate([off.astype(i32) * 2, cid * 2 + 1])
    inv4 = jnp.argsort(jnp.argsort(ck)).astype(i32)
    cnb = inv4[nblk:] - cid - 1
    cnb = jnp.clip(cnb, 0, nblk - 1)
    within = cid - off[cnb]
    cg0 = jnp.clip(cs[cnb] + within, 0, gmax // ngc - 1).astype(i32)
    cf = (within == 0).astype(i32)
    wf = (cid < total_chunks).astype(i32)

    wx = jnp.zeros((XW, H), f32).at[:f].set(W1a[:f])
    we = W1a[f:]
    eye = jnp.eye(GRP, dtype=f32)
    wxbd = jnp.kron(eye, wx).astype(bf)
    webd = jnp.kron(eye, we).astype(bf)
    w1bd = jnp.kron(eye, W1b).astype(bf)
    b1t = jnp.tile(b1a.reshape(1, H), (1, GRP))
    b2t = jnp.tile(b1b.reshape(1, H), (1, GRP))
    vex = jnp.kron(eye, jnp.ones((1, H), f32))
    w2ax = jnp.zeros((XW, 32), f32).at[:f].set(W2a[:f])
    w2aa = W2a[f:]
    b2a2 = b2a.reshape(1, 32)
    w2b8 = jnp.zeros((32, 8), f32).at[:, :4].set(W2b)
    b2b8 = jnp.zeros((1, 8), f32).at[0, :4].set(b2b)

    x_pad = jnp.zeros((n_pad, XW), f32).at[:n, :f].set(x)
    zt = jnp.zeros((epad, XW), f32)
    for _ in range(3):
        xj = _sc_gather(jnp.concatenate([x_pad, zt]), slot_src)
        xg = xj.reshape(gmax, GRP * XW)
        agg = _edge_agg(xg, ea_g, val_g, gn2d, g0, g1, wxbd, webd, b1t,
                        w1bd, b2t, vex, cnb, cg0, cf, wf, n_pad)
        x_pad = _node_update(x_pad, agg, w2ax, w2aa, b2a2, w2b8, b2b8)
    return x_pad[:n, :f]
